# baseline (speedup 1.0000x reference)
import dataclasses
import functools

import jax
import jax.numpy as jnp
from jax import lax
from jax.experimental import pallas as pl
from jax.experimental.pallas import tpu as pltpu
from jax.experimental.pallas import tpu_sc as plsc

VOCAB = 1000000
DIM = 32
BATCH = 1024
WINDOW = 20
NSAMP = 20

NCORES = 2
NSUB = 16
LANES = 16
NWORK = NCORES * NSUB
PAIRS = BATCH * WINDOW
PW = PAIRS // NWORK
CP = 64
NCHUNK = PW // CP
BPW = BATCH // NWORK
COLS = 24

TBLK = 16384


def _sc_compiler_params():
    cp = pltpu.CompilerParams()
    if "needs_layout_passes" in pltpu.CompilerParams.__dataclass_fields__:
        cp = dataclasses.replace(cp, needs_layout_passes=False)
    if "use_tc_tiling_on_sc" in pltpu.CompilerParams.__dataclass_fields__:
        cp = dataclasses.replace(cp, use_tc_tiling_on_sc=False)
    return cp


def _tc_relayout(table_t):
    def body(x_ref, o_ref):
        eye = jnp.eye(DIM, dtype=jnp.float32)
        o_ref[...] = lax.dot_general(
            x_ref[...], eye, (((0,), (0,)), ((), ())),
            precision=lax.Precision.HIGHEST)

    return pl.pallas_call(
        body,
        grid=(pl.cdiv(VOCAB, TBLK),),
        in_specs=[pl.BlockSpec((DIM, TBLK), lambda i: (0, i))],
        out_specs=pl.BlockSpec((TBLK, DIM), lambda i: (i, 0)),
        out_shape=jax.ShapeDtypeStruct((VOCAB, DIM), jnp.float32),
    )(table_t)


def _sc_scores(inputs_f, labels_f, noise_f, iemb_rm, oemb_rm):
    mesh = plsc.VectorSubcoreMesh(core_axis_name="c", subcore_axis_name="s")

    @functools.partial(
        pl.kernel,
        compiler_params=_sc_compiler_params(),
        out_type=jax.ShapeDtypeStruct((PAIRS * COLS,), jnp.float32),
        mesh=mesh,
        scratch_types=[
            pltpu.VMEM((BPW,), jnp.int32),
            pltpu.VMEM((BPW, DIM), jnp.float32),
            pltpu.VMEM((CP,), jnp.int32),
            pltpu.VMEM((CP,), jnp.int32),
            pltpu.VMEM((CP, DIM), jnp.float32),
            pltpu.VMEM((CP, DIM), jnp.float32),
            pltpu.VMEM((CP * NSAMP,), jnp.int32),
            pltpu.VMEM((CP * NSAMP,), jnp.int32),
            pltpu.VMEM((NSAMP, CP), jnp.int32),
            pltpu.VMEM((NSAMP, CP), jnp.int32),
            pltpu.VMEM((NSAMP * CP, DIM), jnp.float32),
            pltpu.VMEM((NSAMP * CP, DIM), jnp.float32),
            pltpu.VMEM((CP * COLS,), jnp.float32),
            pltpu.VMEM((CP * COLS,), jnp.float32),
            pltpu.SemaphoreType.DMA,
            pltpu.SemaphoreType.DMA,
            pltpu.SemaphoreType.DMA,
        ],
    )
    def kern(inputs_hbm, labels_hbm, noise_hbm, iemb_hbm, oemb_hbm, scores_hbm,
             binp_idx, inp_rows, lab0, lab1, out0, out1, nlin0, nlin1,
             nt0, nt1, nr0, nr1, sc0, sc1, sem0, sem1, semi):
        lab = (lab0, lab1)
        outr = (out0, out1)
        nlin = (nlin0, nlin1)
        nt = (nt0, nt1)
        nrows = (nr0, nr1)
        scv = (sc0, sc1)
        sems = (sem0, sem1)

        wid = lax.axis_index("s") * NCORES + lax.axis_index("c")
        wp0 = wid * PW
        iota = lax.iota(jnp.int32, LANES)

        pltpu.sync_copy(inputs_hbm.at[pl.ds(wid * BPW, BPW)], binp_idx)
        pltpu.async_copy(iemb_hbm.at[binp_idx], inp_rows, semi).wait()

        def stage(c, bi):
            bp = wp0 + c * CP
            pltpu.sync_copy(labels_hbm.at[pl.ds(bp, CP)], lab[bi])
            pltpu.sync_copy(noise_hbm.at[pl.ds(bp * NSAMP, CP * NSAMP)],
                            nlin[bi])
            for g in range(CP // LANES):
                rowbase = (iota + g * LANES) * NSAMP
                for s in range(NSAMP):
                    v = plsc.load_gather(nlin[bi], [rowbase + s])
                    nt[bi][s, pl.ds(g * LANES, LANES)] = v
            pltpu.async_copy(oemb_hbm.at[lab[bi]], outr[bi], sems[bi])
            for s in range(NSAMP):
                pltpu.async_copy(oemb_hbm.at[nt[bi].at[s]],
                                 nrows[bi].at[pl.ds(s * CP, CP)], sems[bi])

        def wait_chunk(bi):
            pltpu.make_async_copy(oemb_hbm.at[pl.ds(0, CP)], outr[bi],
                                  sems[bi]).wait()
            pltpu.make_async_copy(oemb_hbm.at[pl.ds(0, NSAMP * CP)],
                                  nrows[bi], sems[bi]).wait()

        def compute(c, bi):
            def group(g, carry):
                pch = iota + g * LANES
                bloc = (pch + c * CP) // WINDOW
                dvecs = [jnp.full((LANES,), d, jnp.int32) for d in range(DIM)]
                inpv = [plsc.load_gather(inp_rows, [bloc, dvecs[d]])
                        for d in range(DIM)]
                base = pch * COLS
                acc = jnp.zeros((LANES,), jnp.float32)
                for d in range(DIM):
                    acc = acc + inpv[d] * plsc.load_gather(
                        outr[bi], [pch, dvecs[d]])
                plsc.store_scatter(scv[bi], [base + NSAMP], acc)
                for s in range(NSAMP):
                    rowv = pch + s * CP
                    acc = jnp.zeros((LANES,), jnp.float32)
                    for d in range(DIM):
                        acc = acc - inpv[d] * plsc.load_gather(
                            nrows[bi], [rowv, dvecs[d]])
                    plsc.store_scatter(scv[bi], [base + s], acc)
                zero = jnp.zeros((LANES,), jnp.float32)
                for pcol in range(NSAMP + 1, COLS):
                    plsc.store_scatter(scv[bi], [base + pcol], zero)
                return carry

            lax.fori_loop(0, CP // LANES, group, 0)

        stage(0, 0)
        stage(1, 1)

        @pl.loop(0, NCHUNK, step=2)
        def _(c):
            for par in range(2):
                cc = c + par
                wait_chunk(par)
                compute(cc, par)
                pltpu.sync_copy(scv[par],
                                scores_hbm.at[pl.ds((wp0 + cc * CP) * COLS,
                                                    CP * COLS)])

                @pl.when(cc + 2 < NCHUNK)
                def _():
                    stage(cc + 2, par)

    return kern(inputs_f, labels_f, noise_f, iemb_rm, oemb_rm)


def _tc_loss(scores):
    rows = PAIRS * COLS // 128
    x2 = scores.reshape(rows, 128)

    def body(s_ref, o_ref):
        x = s_ref[...]
        r = lax.broadcasted_iota(jnp.int32, x.shape, 0)
        cc = lax.broadcasted_iota(jnp.int32, x.shape, 1)
        j = (r * 128 + cc) % COLS
        val = jnp.where(j <= NSAMP, jnp.log(jax.nn.sigmoid(x)), 0.0)
        o_ref[0, 0] = -jnp.sum(val) / BATCH

    out = pl.pallas_call(
        body,
        out_shape=jax.ShapeDtypeStruct((1, 1), jnp.float32),
        out_specs=pl.BlockSpec(memory_space=pltpu.SMEM),
    )(x2)
    return out[0, 0]


def kernel(inputs, labels, num_sampled, input_embed, out_embed, noise_idx):
    inputs_f = inputs.reshape(-1).astype(jnp.int32)
    labels_f = labels.reshape(-1).astype(jnp.int32)
    noise_f = noise_idx.reshape(-1).astype(jnp.int32)
    iemb_rm = _tc_relayout(jnp.transpose(input_embed))
    oemb_rm = _tc_relayout(jnp.transpose(out_embed))
    scores = _sc_scores(inputs_f, labels_f, noise_f, iemb_rm, oemb_rm)
    return _tc_loss(scores)


# speedup vs baseline: 1.2956x; 1.2956x over previous
import dataclasses
import functools

import jax
import jax.numpy as jnp
from jax import lax
from jax.experimental import pallas as pl
from jax.experimental.pallas import tpu as pltpu
from jax.experimental.pallas import tpu_sc as plsc

VOCAB = 1000000
DIM = 32
BATCH = 1024
WINDOW = 20
NSAMP = 20

NCORES = 2
NSUB = 16
LANES = 16
NWORK = NCORES * NSUB
PAIRS = BATCH * WINDOW
PW = PAIRS // NWORK
CP = 80
NCHUNK = PW // CP
BPW = BATCH // NWORK
COLS = 24

WWIN = 256
NWIN = (VOCAB - 64) // WWIN
TAIL0 = NWIN * WWIN


def _sc_compiler_params(tc_tiling):
    cp = pltpu.CompilerParams()
    if "needs_layout_passes" in pltpu.CompilerParams.__dataclass_fields__:
        cp = dataclasses.replace(cp, needs_layout_passes=False)
    if "use_tc_tiling_on_sc" in pltpu.CompilerParams.__dataclass_fields__:
        cp = dataclasses.replace(cp, use_tc_tiling_on_sc=tc_tiling)
    return cp


def _sc_prep(iemb_t, oemb_t, itail, otail, inputs_f):
    mesh = plsc.VectorSubcoreMesh(core_axis_name="c", subcore_axis_name="s")

    @functools.partial(
        pl.kernel,
        compiler_params=_sc_compiler_params(True),
        out_type=(jax.ShapeDtypeStruct((VOCAB, DIM), jnp.float32),
                  jax.ShapeDtypeStruct((BATCH, DIM), jnp.float32)),
        mesh=mesh,
        scratch_types=[
            pltpu.VMEM((DIM, WWIN), jnp.float32),
            pltpu.VMEM((DIM, WWIN), jnp.float32),
            pltpu.VMEM((WWIN, DIM), jnp.float32),
            pltpu.VMEM((DIM, 64), jnp.float32),
            pltpu.VMEM((64, DIM), jnp.float32),
            pltpu.VMEM((BPW, DIM), jnp.float32),
            pltpu.VMEM((BPW,), jnp.int32),
            pltpu.SemaphoreType.DMA,
            pltpu.SemaphoreType.DMA,
        ],
    )
    def kern(iemb_hbm, oemb_hbm, itail_hbm, otail_hbm, inputs_hbm,
             rm_hbm, irows_hbm,
             tv0, tv1, ov, tt, ov64, ivals, idxs, sem0, sem1):
        tv = (tv0, tv1)
        sems = (sem0, sem1)
        wid = lax.axis_index("s") * NCORES + lax.axis_index("c")
        iota = lax.iota(jnp.int32, LANES)
        dvecs = [jnp.full((LANES,), d, jnp.int32) for d in range(DIM)]

        def transpose_win(src, width, dst):
            for g in range(width // LANES):
                lanev = iota + g * LANES
                for d in range(DIM):
                    v = src[d, pl.ds(g * LANES, LANES)]
                    plsc.store_scatter(dst, [lanev, dvecs[d]], v)

        pltpu.sync_copy(inputs_hbm.at[pl.ds(wid * BPW, BPW)], idxs)

        def iread(i):
            s16 = pl.multiple_of((i // LANES) * LANES, 8)
            v = idxs[pl.ds(s16, LANES)]
            return jnp.sum(jnp.where(iota == i % LANES, v, 0))

        def ifire(i, par):
            r = iread(i)
            base = jnp.minimum((r // 128) * 128, TAIL0 - WWIN + 64)
            base = pl.multiple_of(base, 128)
            pltpu.async_copy(iemb_hbm.at[:, pl.ds(base, WWIN)],
                             tv[par], sems[par])

        def idrain(par):
            pltpu.make_async_copy(iemb_hbm.at[:, pl.ds(0, WWIN)],
                                  tv[par], sems[par]).wait()

        ifire(0, 0)
        ifire(1, 1)

        @pl.loop(0, BPW, step=2)
        def _(c):
            for par in range(2):
                i = c + par
                idrain(par)
                r = iread(i)
                isplat = jnp.full((LANES,), 0, jnp.int32) + i

                @pl.when(r < TAIL0)
                def _():
                    base = jnp.minimum((r // 128) * 128, TAIL0 - WWIN + 64)
                    lane = jnp.full((LANES,), 0, jnp.int32) + (r - base)
                    for h in range(2):
                        v = plsc.load_gather(
                            tv[par], [iota + h * LANES, lane])
                        plsc.store_scatter(ivals, [isplat, iota + h * LANES],
                                           v)

                @pl.when(r >= TAIL0)
                def _():
                    pltpu.sync_copy(itail_hbm, tt)
                    lane = jnp.full((LANES,), 0, jnp.int32) + (r - TAIL0)
                    for h in range(2):
                        v = plsc.load_gather(tt, [iota + h * LANES, lane])
                        plsc.store_scatter(ivals, [isplat, iota + h * LANES],
                                           v)

                @pl.when(i + 2 < BPW)
                def _():
                    ifire(i + 2, par)

        pltpu.sync_copy(ivals, irows_hbm.at[pl.ds(wid * BPW, BPW)])

        nw = jnp.where(wid < 2, 123, 122)
        lo = wid * 122 + jnp.minimum(wid, 2)

        def wfire(k, par):
            j = lo + k
            pltpu.async_copy(oemb_hbm.at[:, pl.ds(j * WWIN, WWIN)],
                             tv[par], sems[par])

        def wdrain(par):
            pltpu.make_async_copy(oemb_hbm.at[:, pl.ds(0, WWIN)],
                                  tv[par], sems[par]).wait()

        @pl.when(0 < nw)
        def _():
            wfire(0, 0)

        @pl.when(1 < nw)
        def _():
            wfire(1, 1)

        @pl.loop(0, 124, step=2)
        def _(c):
            for par in range(2):
                k = c + par

                @pl.when(k < nw)
                def _():
                    wdrain(par)
                    transpose_win(tv[par], WWIN, ov)
                    j = lo + k
                    pltpu.sync_copy(ov, rm_hbm.at[pl.ds(j * WWIN, WWIN)])

                    @pl.when(k + 2 < nw)
                    def _():
                        wfire(k + 2, par)

        @pl.when(wid == NWORK - 1)
        def _():
            pltpu.sync_copy(otail_hbm, tt)
            for g in range(64 // LANES):
                lanev = iota + g * LANES
                for d in range(DIM):
                    v = tt[d, pl.ds(g * LANES, LANES)]
                    plsc.store_scatter(ov64, [lanev, dvecs[d]], v)
            pltpu.sync_copy(ov64, rm_hbm.at[pl.ds(TAIL0, 64)])

    return kern(iemb_t, oemb_t, itail, otail, inputs_f)


def _sc_scores(labels_f, noise_f, irows, oemb_rm):
    mesh = plsc.VectorSubcoreMesh(core_axis_name="c", subcore_axis_name="s")

    @functools.partial(
        pl.kernel,
        compiler_params=_sc_compiler_params(False),
        out_type=jax.ShapeDtypeStruct((PAIRS * COLS,), jnp.float32),
        mesh=mesh,
        scratch_types=[
            pltpu.VMEM((BPW, DIM), jnp.float32),
            pltpu.VMEM((CP,), jnp.int32),
            pltpu.VMEM((CP,), jnp.int32),
            pltpu.VMEM((CP, DIM), jnp.float32),
            pltpu.VMEM((CP, DIM), jnp.float32),
            pltpu.VMEM((CP * NSAMP,), jnp.int32),
            pltpu.VMEM((CP * NSAMP,), jnp.int32),
            pltpu.VMEM((NSAMP, CP), jnp.int32),
            pltpu.VMEM((NSAMP, CP), jnp.int32),
            pltpu.VMEM((NSAMP, CP, DIM), jnp.float32),
            pltpu.VMEM((NSAMP, CP, DIM), jnp.float32),
            pltpu.VMEM((CP * COLS,), jnp.float32),
            pltpu.VMEM((CP * COLS,), jnp.float32),
            pltpu.SemaphoreType.DMA,
            pltpu.SemaphoreType.DMA,
        ],
    )
    def kern(labels_hbm, noise_hbm, irows_hbm, oemb_hbm, scores_hbm,
             inp_rows, lab0, lab1, out0, out1, nlin0, nlin1,
             nt0, nt1, nr0, nr1, sc0, sc1, sem0, sem1):
        lab = (lab0, lab1)
        outr = (out0, out1)
        nlin = (nlin0, nlin1)
        nt = (nt0, nt1)
        nrows = (nr0, nr1)
        scv = (sc0, sc1)
        sems = (sem0, sem1)

        wid = lax.axis_index("s") * NCORES + lax.axis_index("c")
        wp0 = wid * PW
        iota = lax.iota(jnp.int32, LANES)

        pltpu.sync_copy(irows_hbm.at[pl.ds(wid * BPW, BPW)], inp_rows)

        def stage(c, bi):
            bp = wp0 + c * CP
            pltpu.sync_copy(labels_hbm.at[pl.ds(bp, CP)], lab[bi])
            pltpu.sync_copy(noise_hbm.at[pl.ds(bp * NSAMP, CP * NSAMP)],
                            nlin[bi])
            for g in range(CP // LANES):
                rowbase = (iota + g * LANES) * NSAMP
                for s in range(NSAMP):
                    v = plsc.load_gather(nlin[bi], [rowbase + s])
                    nt[bi][s, pl.ds(g * LANES, LANES)] = v
            pltpu.async_copy(oemb_hbm.at[lab[bi]], outr[bi], sems[bi])
            for s in range(NSAMP):
                pltpu.async_copy(oemb_hbm.at[nt[bi].at[s]],
                                 nrows[bi].at[s], sems[bi])

        def wait_chunk(bi):
            pltpu.make_async_copy(oemb_hbm.at[pl.ds(0, CP)], outr[bi],
                                  sems[bi]).wait()
            for s in range(NSAMP):
                pltpu.make_async_copy(oemb_hbm.at[pl.ds(0, CP)],
                                      nrows[bi].at[s], sems[bi]).wait()

        def compute(c, bi):
            def group(g, carry):
                pch = iota + g * LANES
                bloc = (pch + c * CP) // WINDOW
                dvecs = [jnp.full((LANES,), d, jnp.int32) for d in range(DIM)]
                svecs = [jnp.full((LANES,), s, jnp.int32)
                         for s in range(NSAMP)]
                inpv = [plsc.load_gather(inp_rows, [bloc, dvecs[d]])
                        for d in range(DIM)]
                base = pch * COLS
                acc = jnp.zeros((LANES,), jnp.float32)
                for d in range(DIM):
                    acc = acc + inpv[d] * plsc.load_gather(
                        outr[bi], [pch, dvecs[d]])
                plsc.store_scatter(scv[bi], [base + NSAMP], acc)
                for s in range(NSAMP):
                    acc = jnp.zeros((LANES,), jnp.float32)
                    for d in range(DIM):
                        acc = acc - inpv[d] * plsc.load_gather(
                            nrows[bi], [svecs[s], pch, dvecs[d]])
                    plsc.store_scatter(scv[bi], [base + s], acc)
                zero = jnp.zeros((LANES,), jnp.float32)
                for pcol in range(NSAMP + 1, COLS):
                    plsc.store_scatter(scv[bi], [base + pcol], zero)
                return carry

            lax.fori_loop(0, CP // LANES, group, 0)

        stage(0, 0)
        stage(1, 1)

        @pl.loop(0, NCHUNK, step=2)
        def _(c):
            for par in range(2):
                cc = c + par
                wait_chunk(par)
                compute(cc, par)
                pltpu.sync_copy(scv[par],
                                scores_hbm.at[pl.ds((wp0 + cc * CP) * COLS,
                                                    CP * COLS)])

                @pl.when(cc + 2 < NCHUNK)
                def _():
                    stage(cc + 2, par)

    return kern(labels_f, noise_f, irows, oemb_rm)


def _tc_loss(scores):
    rows = PAIRS * COLS // 128
    x2 = scores.reshape(rows, 128)

    def body(s_ref, o_ref):
        x = s_ref[...]
        r = lax.broadcasted_iota(jnp.int32, x.shape, 0)
        cc = lax.broadcasted_iota(jnp.int32, x.shape, 1)
        j = (r * 128 + cc) % COLS
        val = jnp.where(j <= NSAMP, jnp.log(jax.nn.sigmoid(x)), 0.0)
        o_ref[0, 0] = -jnp.sum(val) / BATCH

    out = pl.pallas_call(
        body,
        out_shape=jax.ShapeDtypeStruct((1, 1), jnp.float32),
        out_specs=pl.BlockSpec(memory_space=pltpu.SMEM),
    )(x2)
    return out[0, 0]


def kernel(inputs, labels, num_sampled, input_embed, out_embed, noise_idx):
    inputs_f = inputs.reshape(-1).astype(jnp.int32)
    labels_f = labels.reshape(-1).astype(jnp.int32)
    noise_f = noise_idx.reshape(-1).astype(jnp.int32)
    iemb_t = jnp.transpose(input_embed)
    oemb_t = jnp.transpose(out_embed)
    itail = lax.slice(iemb_t, (0, TAIL0), (DIM, VOCAB))
    otail = lax.slice(oemb_t, (0, TAIL0), (DIM, VOCAB))
    oemb_rm, irows = _sc_prep(iemb_t, oemb_t, itail, otail, inputs_f)
    scores = _sc_scores(labels_f, noise_f, irows, oemb_rm)
    return _tc_loss(scores)


# speedup vs baseline: 1.9150x; 1.4781x over previous
import dataclasses
import functools

import jax
import jax.numpy as jnp
from jax import lax
from jax.experimental import pallas as pl
from jax.experimental.pallas import tpu as pltpu
from jax.experimental.pallas import tpu_sc as plsc

VOCAB = 1000000
DIM = 32
BATCH = 1024
WINDOW = 20
NSAMP = 20

NCORES = 2
NSUB = 16
LANES = 16
NWORK = NCORES * NSUB
PAIRS = BATCH * WINDOW
PW = PAIRS // NWORK
CP = 64
NCHUNK = PW // CP
GROWS = 128
NGATH = CP * NSAMP // GROWS
BPW = BATCH // NWORK
COLS = 24

WWIN = 256
NWIN = (VOCAB - 64) // WWIN
TAIL0 = NWIN * WWIN


def _sc_compiler_params(tc_tiling):
    cp = pltpu.CompilerParams()
    if "needs_layout_passes" in pltpu.CompilerParams.__dataclass_fields__:
        cp = dataclasses.replace(cp, needs_layout_passes=False)
    if "use_tc_tiling_on_sc" in pltpu.CompilerParams.__dataclass_fields__:
        cp = dataclasses.replace(cp, use_tc_tiling_on_sc=tc_tiling)
    return cp


def _sc_prep(iemb_t, oemb_t, itail, otail, inputs_f):
    mesh = plsc.VectorSubcoreMesh(core_axis_name="c", subcore_axis_name="s")

    @functools.partial(
        pl.kernel,
        compiler_params=_sc_compiler_params(True),
        out_type=(jax.ShapeDtypeStruct((VOCAB * DIM,), jnp.float32),
                  jax.ShapeDtypeStruct((BATCH * DIM,), jnp.float32)),
        mesh=mesh,
        scratch_types=[
            pltpu.VMEM((DIM, WWIN), jnp.float32),
            pltpu.VMEM((DIM, WWIN), jnp.float32),
            pltpu.VMEM((WWIN * DIM,), jnp.float32),
            pltpu.VMEM((DIM, 64), jnp.float32),
            pltpu.VMEM((64 * DIM,), jnp.float32),
            pltpu.VMEM((BPW * DIM,), jnp.float32),
            pltpu.VMEM((BPW,), jnp.int32),
            pltpu.SemaphoreType.DMA,
            pltpu.SemaphoreType.DMA,
        ],
    )
    def kern(iemb_hbm, oemb_hbm, itail_hbm, otail_hbm, inputs_hbm,
             rm_hbm, irows_hbm,
             tv0, tv1, ov, tt, ov64, ivals, idxs, sem0, sem1):
        tv = (tv0, tv1)
        sems = (sem0, sem1)
        wid = lax.axis_index("s") * NCORES + lax.axis_index("c")
        iota = lax.iota(jnp.int32, LANES)

        def transpose_win(src, width, dst):
            for g in range(width // LANES):
                lanev_d = (iota + g * LANES) * DIM
                for d in range(DIM):
                    v = src[d, pl.ds(g * LANES, LANES)]
                    plsc.store_scatter(dst, [lanev_d + d], v)

        pltpu.sync_copy(inputs_hbm.at[pl.ds(wid * BPW, BPW)], idxs)

        def iread(i):
            s16 = pl.multiple_of((i // LANES) * LANES, 8)
            v = idxs[pl.ds(s16, LANES)]
            return jnp.sum(jnp.where(iota == i % LANES, v, 0))

        def ifire(i, par):
            r = iread(i)
            base = jnp.minimum((r // 128) * 128, TAIL0 - WWIN + 64)
            base = pl.multiple_of(base, 128)
            pltpu.async_copy(iemb_hbm.at[:, pl.ds(base, WWIN)],
                             tv[par], sems[par])

        def idrain(par):
            pltpu.make_async_copy(iemb_hbm.at[:, pl.ds(0, WWIN)],
                                  tv[par], sems[par]).wait()

        ifire(0, 0)
        ifire(1, 1)

        @pl.loop(0, BPW, step=2)
        def _(c):
            for par in range(2):
                i = c + par
                idrain(par)
                r = iread(i)
                ibase = i * DIM

                @pl.when(r < TAIL0)
                def _():
                    base = jnp.minimum((r // 128) * 128, TAIL0 - WWIN + 64)
                    lane = jnp.full((LANES,), 0, jnp.int32) + (r - base)
                    for h in range(2):
                        v = plsc.load_gather(
                            tv[par], [iota + h * LANES, lane])
                        plsc.store_scatter(
                            ivals, [(iota + h * LANES) + ibase], v)

                @pl.when(r >= TAIL0)
                def _():
                    pltpu.sync_copy(itail_hbm, tt)
                    lane = jnp.full((LANES,), 0, jnp.int32) + (r - TAIL0)
                    for h in range(2):
                        v = plsc.load_gather(tt, [iota + h * LANES, lane])
                        plsc.store_scatter(
                            ivals, [(iota + h * LANES) + ibase], v)

                @pl.when(i + 2 < BPW)
                def _():
                    ifire(i + 2, par)

        pltpu.sync_copy(ivals,
                        irows_hbm.at[pl.ds(wid * (BPW * DIM), BPW * DIM)])

        nw = jnp.where(wid < 2, 123, 122)
        lo = wid * 122 + jnp.minimum(wid, 2)

        def wfire(k, par):
            j = lo + k
            pltpu.async_copy(oemb_hbm.at[:, pl.ds(j * WWIN, WWIN)],
                             tv[par], sems[par])

        def wdrain(par):
            pltpu.make_async_copy(oemb_hbm.at[:, pl.ds(0, WWIN)],
                                  tv[par], sems[par]).wait()

        @pl.when(0 < nw)
        def _():
            wfire(0, 0)

        @pl.when(1 < nw)
        def _():
            wfire(1, 1)

        @pl.loop(0, 124, step=2)
        def _(c):
            for par in range(2):
                k = c + par

                @pl.when(k < nw)
                def _():
                    wdrain(par)
                    transpose_win(tv[par], WWIN, ov)
                    j = lo + k
                    pltpu.sync_copy(
                        ov, rm_hbm.at[pl.ds(j * (WWIN * DIM), WWIN * DIM)])

                    @pl.when(k + 2 < nw)
                    def _():
                        wfire(k + 2, par)

        @pl.when(wid == NWORK - 1)
        def _():
            pltpu.sync_copy(otail_hbm, tt)
            transpose_win(tt, 64, ov64)
            pltpu.sync_copy(ov64, rm_hbm.at[pl.ds(TAIL0 * DIM, 64 * DIM)])

    return kern(iemb_t, oemb_t, itail, otail, inputs_f)


def _sc_scores(labels_f, noise_f, irows, oemb_rm):
    mesh = plsc.VectorSubcoreMesh(core_axis_name="c", subcore_axis_name="s")

    @functools.partial(
        pl.kernel,
        compiler_params=_sc_compiler_params(False),
        out_type=jax.ShapeDtypeStruct((PAIRS * COLS,), jnp.float32),
        mesh=mesh,
        scratch_types=[
            pltpu.VMEM((BPW * DIM,), jnp.float32),
            pltpu.VMEM((CP,), jnp.int32),
            pltpu.VMEM((CP,), jnp.int32),
            pltpu.VMEM((CP, DIM), jnp.float32),
            pltpu.VMEM((CP, DIM), jnp.float32),
            pltpu.VMEM((CP * NSAMP,), jnp.int32),
            pltpu.VMEM((CP * NSAMP,), jnp.int32),
            pltpu.VMEM((NSAMP * CP,), jnp.int32),
            pltpu.VMEM((NSAMP * CP,), jnp.int32),
            pltpu.VMEM((NSAMP * CP, DIM), jnp.float32),
            pltpu.VMEM((NSAMP * CP, DIM), jnp.float32),
            pltpu.VMEM((CP * COLS,), jnp.float32),
            pltpu.VMEM((CP * COLS,), jnp.float32),
            pltpu.SemaphoreType.DMA,
            pltpu.SemaphoreType.DMA,
        ],
    )
    def kern(labels_hbm, noise_hbm, irows_hbm, oemb_hbm, scores_hbm,
             inp_rows, lab0, lab1, out0, out1, nlin0, nlin1,
             nt0, nt1, nr0, nr1, sc0, sc1, sem0, sem1):
        lab = (lab0, lab1)
        outr = (out0, out1)
        nlin = (nlin0, nlin1)
        nt = (nt0, nt1)
        nrows = (nr0, nr1)
        scv = (sc0, sc1)
        sems = (sem0, sem1)

        wid = lax.axis_index("s") * NCORES + lax.axis_index("c")
        wp0 = wid * PW
        iota = lax.iota(jnp.int32, LANES)

        pltpu.sync_copy(irows_hbm.at[pl.ds(wid * (BPW * DIM), BPW * DIM)],
                        inp_rows)

        def stage(c, bi):
            bp = wp0 + c * CP
            pltpu.sync_copy(labels_hbm.at[pl.ds(bp, CP)], lab[bi])
            pltpu.sync_copy(noise_hbm.at[pl.ds(bp * NSAMP, CP * NSAMP)],
                            nlin[bi])
            for g in range(CP // LANES):
                rowbase = (iota + g * LANES) * NSAMP
                for s in range(NSAMP):
                    v = plsc.load_gather(nlin[bi], [rowbase + s])
                    nt[bi][pl.ds(s * CP + g * LANES, LANES)] = v
            pltpu.async_copy(oemb_hbm.at[lab[bi]], outr[bi], sems[bi])
            for k in range(NGATH):
                pltpu.async_copy(
                    oemb_hbm.at[nt[bi].at[pl.ds(k * GROWS, GROWS)]],
                    nrows[bi].at[pl.ds(k * GROWS, GROWS)], sems[bi])

        def wait_chunk(bi):
            pltpu.make_async_copy(oemb_hbm.at[pl.ds(0, CP)], outr[bi],
                                  sems[bi]).wait()
            for k in range(NGATH):
                pltpu.make_async_copy(
                    oemb_hbm.at[pl.ds(0, GROWS)],
                    nrows[bi].at[pl.ds(k * GROWS, GROWS)], sems[bi]).wait()

        def compute(c, bi):
            def group(g, carry):
                pch = iota + g * LANES
                bloc = (pch + c * CP) // WINDOW
                dvecs = [jnp.full((LANES,), d, jnp.int32) for d in range(DIM)]
                inpv = [plsc.load_gather(inp_rows, [bloc * DIM + d])
                        for d in range(DIM)]
                base = pch * COLS
                acc = jnp.zeros((LANES,), jnp.float32)
                for d in range(DIM):
                    acc = acc + inpv[d] * plsc.load_gather(
                        outr[bi], [pch, dvecs[d]])
                plsc.store_scatter(scv[bi], [base + NSAMP], acc)
                for s in range(NSAMP):
                    rowv = pch + s * CP
                    acc = jnp.zeros((LANES,), jnp.float32)
                    for d in range(DIM):
                        acc = acc - inpv[d] * plsc.load_gather(
                            nrows[bi], [rowv, dvecs[d]])
                    plsc.store_scatter(scv[bi], [base + s], acc)
                zero = jnp.zeros((LANES,), jnp.float32)
                for pcol in range(NSAMP + 1, COLS):
                    plsc.store_scatter(scv[bi], [base + pcol], zero)
                return carry

            lax.fori_loop(0, CP // LANES, group, 0)

        stage(0, 0)
        stage(1, 1)

        @pl.loop(0, NCHUNK, step=2)
        def _(c):
            for par in range(2):
                cc = c + par
                wait_chunk(par)
                compute(cc, par)
                pltpu.sync_copy(scv[par],
                                scores_hbm.at[pl.ds((wp0 + cc * CP) * COLS,
                                                    CP * COLS)])

                @pl.when(cc + 2 < NCHUNK)
                def _():
                    stage(cc + 2, par)

    return kern(labels_f, noise_f, irows, oemb_rm)


def _tc_loss(scores):
    rows = PAIRS * COLS // 128
    x2 = scores.reshape(rows, 128)

    def body(s_ref, o_ref):
        x = s_ref[...]
        r = lax.broadcasted_iota(jnp.int32, x.shape, 0)
        cc = lax.broadcasted_iota(jnp.int32, x.shape, 1)
        j = (r * 128 + cc) % COLS
        val = jnp.where(j <= NSAMP, jnp.log(jax.nn.sigmoid(x)), 0.0)
        o_ref[0, 0] = -jnp.sum(val) / BATCH

    out = pl.pallas_call(
        body,
        out_shape=jax.ShapeDtypeStruct((1, 1), jnp.float32),
        out_specs=pl.BlockSpec(memory_space=pltpu.SMEM),
    )(x2)
    return out[0, 0]


def kernel(inputs, labels, num_sampled, input_embed, out_embed, noise_idx):
    inputs_f = inputs.reshape(-1).astype(jnp.int32)
    labels_f = labels.reshape(-1).astype(jnp.int32)
    noise_f = noise_idx.reshape(-1).astype(jnp.int32)
    iemb_t = jnp.transpose(input_embed)
    oemb_t = jnp.transpose(out_embed)
    itail = lax.slice(iemb_t, (0, TAIL0), (DIM, VOCAB))
    otail = lax.slice(oemb_t, (0, TAIL0), (DIM, VOCAB))
    oemb_rm, irows = _sc_prep(iemb_t, oemb_t, itail, otail, inputs_f)
    scores = _sc_scores(labels_f, noise_f, irows,
                        oemb_rm.reshape(VOCAB, DIM))
    return _tc_loss(scores)


# speedup vs baseline: 2.2783x; 1.1897x over previous
import dataclasses
import functools

import jax
import jax.numpy as jnp
from jax import lax
from jax.experimental import pallas as pl
from jax.experimental.pallas import tpu as pltpu
from jax.experimental.pallas import tpu_sc as plsc

VOCAB = 1000000
DIM = 32
BATCH = 1024
WINDOW = 20
NSAMP = 20

NCORES = 2
NSUB = 16
LANES = 16
NWORK = NCORES * NSUB
PAIRS = BATCH * WINDOW
PW = PAIRS // NWORK
CP = 64
NCHUNK = PW // CP
GROWS = 128
NGATH = CP * NSAMP // GROWS
BPW = BATCH // NWORK
COLS = 24

WWIN = 256
NWIN = (VOCAB - 64) // WWIN
TAIL0 = NWIN * WWIN


def _sc_compiler_params(tc_tiling):
    cp = pltpu.CompilerParams()
    if "needs_layout_passes" in pltpu.CompilerParams.__dataclass_fields__:
        cp = dataclasses.replace(cp, needs_layout_passes=False)
    if "use_tc_tiling_on_sc" in pltpu.CompilerParams.__dataclass_fields__:
        cp = dataclasses.replace(cp, use_tc_tiling_on_sc=tc_tiling)
    return cp


def _sc_prep(iemb_t, oemb_t, itail, otail, inputs_f):
    mesh = plsc.VectorSubcoreMesh(core_axis_name="c", subcore_axis_name="s")

    @functools.partial(
        pl.kernel,
        compiler_params=_sc_compiler_params(True),
        out_type=(jax.ShapeDtypeStruct((VOCAB * DIM,), jnp.float32),
                  jax.ShapeDtypeStruct((BATCH * DIM,), jnp.float32)),
        mesh=mesh,
        scratch_types=[
            pltpu.VMEM((DIM, WWIN), jnp.float32),
            pltpu.VMEM((DIM, WWIN), jnp.float32),
            pltpu.VMEM((WWIN * DIM,), jnp.float32),
            pltpu.VMEM((DIM, 64), jnp.float32),
            pltpu.VMEM((64 * DIM,), jnp.float32),
            pltpu.VMEM((BPW * DIM,), jnp.float32),
            pltpu.VMEM((BPW,), jnp.int32),
            pltpu.SemaphoreType.DMA,
            pltpu.SemaphoreType.DMA,
        ],
    )
    def kern(iemb_hbm, oemb_hbm, itail_hbm, otail_hbm, inputs_hbm,
             rm_hbm, irows_hbm,
             tv0, tv1, ov, tt, ov64, ivals, idxs, sem0, sem1):
        tv = (tv0, tv1)
        sems = (sem0, sem1)
        wid = lax.axis_index("s") * NCORES + lax.axis_index("c")
        iota = lax.iota(jnp.int32, LANES)

        def transpose_win(src, width, dst):
            for g in range(width // LANES):
                lanev_d = (iota + g * LANES) * DIM
                for db in range(0, DIM, 8):
                    vs = [src[d, pl.ds(g * LANES, LANES)]
                          for d in range(db, db + 8)]
                    for t, d in enumerate(range(db, db + 8)):
                        plsc.store_scatter(dst, [lanev_d + d], vs[t])

        pltpu.sync_copy(inputs_hbm.at[pl.ds(wid * BPW, BPW)], idxs)

        def iread(i):
            s16 = pl.multiple_of((i // LANES) * LANES, 8)
            v = idxs[pl.ds(s16, LANES)]
            return jnp.sum(jnp.where(iota == i % LANES, v, 0))

        def ifire(i, par):
            r = iread(i)
            base = jnp.minimum((r // 128) * 128, TAIL0 - WWIN + 64)
            base = pl.multiple_of(base, 128)
            pltpu.async_copy(iemb_hbm.at[:, pl.ds(base, WWIN)],
                             tv[par], sems[par])

        def idrain(par):
            pltpu.make_async_copy(iemb_hbm.at[:, pl.ds(0, WWIN)],
                                  tv[par], sems[par]).wait()

        ifire(0, 0)
        ifire(1, 1)

        @pl.loop(0, BPW, step=2)
        def _(c):
            for par in range(2):
                i = c + par
                idrain(par)
                r = iread(i)
                ibase = i * DIM

                @pl.when(r < TAIL0)
                def _():
                    base = jnp.minimum((r // 128) * 128, TAIL0 - WWIN + 64)
                    lane = jnp.full((LANES,), 0, jnp.int32) + (r - base)
                    for h in range(2):
                        v = plsc.load_gather(
                            tv[par], [iota + h * LANES, lane])
                        plsc.store_scatter(
                            ivals, [(iota + h * LANES) + ibase], v)

                @pl.when(r >= TAIL0)
                def _():
                    pltpu.sync_copy(itail_hbm, tt)
                    lane = jnp.full((LANES,), 0, jnp.int32) + (r - TAIL0)
                    for h in range(2):
                        v = plsc.load_gather(tt, [iota + h * LANES, lane])
                        plsc.store_scatter(
                            ivals, [(iota + h * LANES) + ibase], v)

                @pl.when(i + 2 < BPW)
                def _():
                    ifire(i + 2, par)

        pltpu.sync_copy(ivals,
                        irows_hbm.at[pl.ds(wid * (BPW * DIM), BPW * DIM)])

        nw = jnp.where(wid < 2, 123, 122)
        lo = wid * 122 + jnp.minimum(wid, 2)

        def wfire(k, par):
            j = lo + k
            pltpu.async_copy(oemb_hbm.at[:, pl.ds(j * WWIN, WWIN)],
                             tv[par], sems[par])

        def wdrain(par):
            pltpu.make_async_copy(oemb_hbm.at[:, pl.ds(0, WWIN)],
                                  tv[par], sems[par]).wait()

        @pl.when(0 < nw)
        def _():
            wfire(0, 0)

        @pl.when(1 < nw)
        def _():
            wfire(1, 1)

        @pl.loop(0, 124, step=2)
        def _(c):
            for par in range(2):
                k = c + par

                @pl.when(k < nw)
                def _():
                    wdrain(par)
                    transpose_win(tv[par], WWIN, ov)
                    j = lo + k
                    pltpu.sync_copy(
                        ov, rm_hbm.at[pl.ds(j * (WWIN * DIM), WWIN * DIM)])

                    @pl.when(k + 2 < nw)
                    def _():
                        wfire(k + 2, par)

        @pl.when(wid == NWORK - 1)
        def _():
            pltpu.sync_copy(otail_hbm, tt)
            transpose_win(tt, 64, ov64)
            pltpu.sync_copy(ov64, rm_hbm.at[pl.ds(TAIL0 * DIM, 64 * DIM)])

    return kern(iemb_t, oemb_t, itail, otail, inputs_f)


def _sc_scores(labels_f, noise_f, irows, oemb_rm):
    mesh = plsc.VectorSubcoreMesh(core_axis_name="c", subcore_axis_name="s")

    @functools.partial(
        pl.kernel,
        compiler_params=_sc_compiler_params(False),
        out_type=jax.ShapeDtypeStruct((PAIRS * COLS,), jnp.float32),
        mesh=mesh,
        scratch_types=[
            pltpu.VMEM((BPW * DIM,), jnp.float32),
            pltpu.VMEM((CP,), jnp.int32),
            pltpu.VMEM((CP,), jnp.int32),
            pltpu.VMEM((CP, DIM), jnp.float32),
            pltpu.VMEM((CP, DIM), jnp.float32),
            pltpu.VMEM((CP * NSAMP,), jnp.int32),
            pltpu.VMEM((CP * NSAMP,), jnp.int32),
            pltpu.VMEM((NSAMP * CP,), jnp.int32),
            pltpu.VMEM((NSAMP * CP,), jnp.int32),
            pltpu.VMEM((NSAMP * CP, DIM), jnp.float32),
            pltpu.VMEM((NSAMP * CP, DIM), jnp.float32),
            pltpu.VMEM((CP * COLS,), jnp.float32),
            pltpu.VMEM((CP * COLS,), jnp.float32),
            pltpu.SemaphoreType.DMA,
            pltpu.SemaphoreType.DMA,
        ],
    )
    def kern(labels_hbm, noise_hbm, irows_hbm, oemb_hbm, scores_hbm,
             inp_rows, lab0, lab1, out0, out1, nlin0, nlin1,
             nt0, nt1, nr0, nr1, sc0, sc1, sem0, sem1):
        lab = (lab0, lab1)
        outr = (out0, out1)
        nlin = (nlin0, nlin1)
        nt = (nt0, nt1)
        nrows = (nr0, nr1)
        scv = (sc0, sc1)
        sems = (sem0, sem1)

        wid = lax.axis_index("s") * NCORES + lax.axis_index("c")
        wp0 = wid * PW
        iota = lax.iota(jnp.int32, LANES)

        pltpu.sync_copy(irows_hbm.at[pl.ds(wid * (BPW * DIM), BPW * DIM)],
                        inp_rows)

        def stage(c, bi):
            bp = wp0 + c * CP
            pltpu.sync_copy(labels_hbm.at[pl.ds(bp, CP)], lab[bi])
            pltpu.sync_copy(noise_hbm.at[pl.ds(bp * NSAMP, CP * NSAMP)],
                            nlin[bi])
            for g in range(CP // LANES):
                rowbase = (iota + g * LANES) * NSAMP
                for s in range(NSAMP):
                    v = plsc.load_gather(nlin[bi], [rowbase + s])
                    nt[bi][pl.ds(s * CP + g * LANES, LANES)] = v
            pltpu.async_copy(oemb_hbm.at[lab[bi]], outr[bi], sems[bi])
            for k in range(NGATH):
                pltpu.async_copy(
                    oemb_hbm.at[nt[bi].at[pl.ds(k * GROWS, GROWS)]],
                    nrows[bi].at[pl.ds(k * GROWS, GROWS)], sems[bi])

        def wait_chunk(bi):
            pltpu.make_async_copy(oemb_hbm.at[pl.ds(0, CP)], outr[bi],
                                  sems[bi]).wait()
            for k in range(NGATH):
                pltpu.make_async_copy(
                    oemb_hbm.at[pl.ds(0, GROWS)],
                    nrows[bi].at[pl.ds(k * GROWS, GROWS)], sems[bi]).wait()

        def compute(c, bi):
            def group(g, carry):
                pch = iota + g * LANES
                bloc = (pch + c * CP) // WINDOW
                dvecs = [jnp.full((LANES,), d, jnp.int32) for d in range(DIM)]
                inpv = [plsc.load_gather(inp_rows, [bloc * DIM + d])
                        for d in range(DIM)]
                base = pch * COLS

                def dot_rows(ref, rowv):
                    a = [jnp.zeros((LANES,), jnp.float32) for _ in range(4)]
                    for db in range(0, DIM, 8):
                        ls = [plsc.load_gather(ref, [rowv, dvecs[d]])
                              for d in range(db, db + 8)]
                        for t, d in enumerate(range(db, db + 8)):
                            a[t % 4] = a[t % 4] + inpv[d] * ls[t]
                    return (a[0] + a[1]) + (a[2] + a[3])

                plsc.store_scatter(scv[bi], [base + NSAMP],
                                   dot_rows(outr[bi], pch))
                for s in range(NSAMP):
                    plsc.store_scatter(scv[bi], [base + s],
                                       -dot_rows(nrows[bi], pch + s * CP))
                zero = jnp.zeros((LANES,), jnp.float32)
                for pcol in range(NSAMP + 1, COLS):
                    plsc.store_scatter(scv[bi], [base + pcol], zero)
                return carry

            lax.fori_loop(0, CP // LANES, group, 0)

        stage(0, 0)
        stage(1, 1)

        @pl.loop(0, NCHUNK, step=2)
        def _(c):
            for par in range(2):
                cc = c + par
                wait_chunk(par)
                compute(cc, par)
                pltpu.sync_copy(scv[par],
                                scores_hbm.at[pl.ds((wp0 + cc * CP) * COLS,
                                                    CP * COLS)])

                @pl.when(cc + 2 < NCHUNK)
                def _():
                    stage(cc + 2, par)

    return kern(labels_f, noise_f, irows, oemb_rm)


def _tc_loss(scores):
    rows = PAIRS * COLS // 128
    x2 = scores.reshape(rows, 128)

    def body(s_ref, o_ref):
        x = s_ref[...]
        r = lax.broadcasted_iota(jnp.int32, x.shape, 0)
        cc = lax.broadcasted_iota(jnp.int32, x.shape, 1)
        j = (r * 128 + cc) % COLS
        val = jnp.where(j <= NSAMP, jnp.log(jax.nn.sigmoid(x)), 0.0)
        o_ref[0, 0] = -jnp.sum(val) / BATCH

    out = pl.pallas_call(
        body,
        out_shape=jax.ShapeDtypeStruct((1, 1), jnp.float32),
        out_specs=pl.BlockSpec(memory_space=pltpu.SMEM),
    )(x2)
    return out[0, 0]


def kernel(inputs, labels, num_sampled, input_embed, out_embed, noise_idx):
    inputs_f = inputs.reshape(-1).astype(jnp.int32)
    labels_f = labels.reshape(-1).astype(jnp.int32)
    noise_f = noise_idx.reshape(-1).astype(jnp.int32)
    iemb_t = jnp.transpose(input_embed)
    oemb_t = jnp.transpose(out_embed)
    itail = lax.slice(iemb_t, (0, TAIL0), (DIM, VOCAB))
    otail = lax.slice(oemb_t, (0, TAIL0), (DIM, VOCAB))
    oemb_rm, irows = _sc_prep(iemb_t, oemb_t, itail, otail, inputs_f)
    scores = _sc_scores(labels_f, noise_f, irows,
                        oemb_rm.reshape(VOCAB, DIM))
    return _tc_loss(scores)


# speedup vs baseline: 2.2922x; 1.0061x over previous
import dataclasses
import functools

import jax
import jax.numpy as jnp
from jax import lax
from jax.experimental import pallas as pl
from jax.experimental.pallas import tpu as pltpu
from jax.experimental.pallas import tpu_sc as plsc

VOCAB = 1000000
DIM = 32
BATCH = 1024
WINDOW = 20
NSAMP = 20

NCORES = 2
NSUB = 16
LANES = 16
NWORK = NCORES * NSUB
PAIRS = BATCH * WINDOW
PW = PAIRS // NWORK
CP = 64
NCHUNK = PW // CP
GROWS = 128
NGATH = CP * NSAMP // GROWS
BPW = BATCH // NWORK
COLS = 24

WWIN = 256
NWIN = (VOCAB - 64) // WWIN
TAIL0 = NWIN * WWIN


def _sc_compiler_params(tc_tiling):
    cp = pltpu.CompilerParams()
    if "needs_layout_passes" in pltpu.CompilerParams.__dataclass_fields__:
        cp = dataclasses.replace(cp, needs_layout_passes=False)
    if "use_tc_tiling_on_sc" in pltpu.CompilerParams.__dataclass_fields__:
        cp = dataclasses.replace(cp, use_tc_tiling_on_sc=tc_tiling)
    return cp


def _sc_prep(iemb_t, oemb_t, itail, otail, inputs_f):
    mesh = plsc.VectorSubcoreMesh(core_axis_name="c", subcore_axis_name="s")

    @functools.partial(
        pl.kernel,
        compiler_params=_sc_compiler_params(True),
        out_type=(jax.ShapeDtypeStruct((VOCAB * DIM,), jnp.float32),
                  jax.ShapeDtypeStruct((BATCH * DIM,), jnp.float32)),
        mesh=mesh,
        scratch_types=[
            pltpu.VMEM((DIM, WWIN), jnp.float32),
            pltpu.VMEM((DIM, WWIN), jnp.float32),
            pltpu.VMEM((WWIN * DIM,), jnp.float32),
            pltpu.VMEM((DIM, 64), jnp.float32),
            pltpu.VMEM((64 * DIM,), jnp.float32),
            pltpu.VMEM((BPW * DIM,), jnp.float32),
            pltpu.VMEM((BPW,), jnp.int32),
            pltpu.SemaphoreType.DMA,
            pltpu.SemaphoreType.DMA,
        ],
    )
    def kern(iemb_hbm, oemb_hbm, itail_hbm, otail_hbm, inputs_hbm,
             rm_hbm, irows_hbm,
             tv0, tv1, ov, tt, ov64, ivals, idxs, sem0, sem1):
        tv = (tv0, tv1)
        sems = (sem0, sem1)
        wid = lax.axis_index("s") * NCORES + lax.axis_index("c")
        iota = lax.iota(jnp.int32, LANES)

        def transpose_win(src, width, dst):
            @plsc.parallel_loop(0, width // LANES, unroll=2)
            def _(g):
                lanev_d = (iota + g * LANES) * DIM
                for db in range(0, DIM, 8):
                    vs = [src[d, pl.ds(g * LANES, LANES)]
                          for d in range(db, db + 8)]
                    for t, d in enumerate(range(db, db + 8)):
                        plsc.store_scatter(dst, [lanev_d + d], vs[t])

        pltpu.sync_copy(inputs_hbm.at[pl.ds(wid * BPW, BPW)], idxs)

        def iread(i):
            s16 = pl.multiple_of((i // LANES) * LANES, 8)
            v = idxs[pl.ds(s16, LANES)]
            return jnp.sum(jnp.where(iota == i % LANES, v, 0))

        def ifire(i, par):
            r = iread(i)
            base = jnp.minimum((r // 128) * 128, TAIL0 - WWIN + 64)
            base = pl.multiple_of(base, 128)
            pltpu.async_copy(iemb_hbm.at[:, pl.ds(base, WWIN)],
                             tv[par], sems[par])

        def idrain(par):
            pltpu.make_async_copy(iemb_hbm.at[:, pl.ds(0, WWIN)],
                                  tv[par], sems[par]).wait()

        ifire(0, 0)
        ifire(1, 1)

        @pl.loop(0, BPW, step=2)
        def _(c):
            for par in range(2):
                i = c + par
                idrain(par)
                r = iread(i)
                ibase = i * DIM

                @pl.when(r < TAIL0)
                def _():
                    base = jnp.minimum((r // 128) * 128, TAIL0 - WWIN + 64)
                    lane = jnp.full((LANES,), 0, jnp.int32) + (r - base)
                    for h in range(2):
                        v = plsc.load_gather(
                            tv[par], [iota + h * LANES, lane])
                        plsc.store_scatter(
                            ivals, [(iota + h * LANES) + ibase], v)

                @pl.when(r >= TAIL0)
                def _():
                    pltpu.sync_copy(itail_hbm, tt)
                    lane = jnp.full((LANES,), 0, jnp.int32) + (r - TAIL0)
                    for h in range(2):
                        v = plsc.load_gather(tt, [iota + h * LANES, lane])
                        plsc.store_scatter(
                            ivals, [(iota + h * LANES) + ibase], v)

                @pl.when(i + 2 < BPW)
                def _():
                    ifire(i + 2, par)

        pltpu.sync_copy(ivals,
                        irows_hbm.at[pl.ds(wid * (BPW * DIM), BPW * DIM)])

        nw = jnp.where(wid < 2, 123, 122)
        lo = wid * 122 + jnp.minimum(wid, 2)

        def wfire(k, par):
            j = lo + k
            pltpu.async_copy(oemb_hbm.at[:, pl.ds(j * WWIN, WWIN)],
                             tv[par], sems[par])

        def wdrain(par):
            pltpu.make_async_copy(oemb_hbm.at[:, pl.ds(0, WWIN)],
                                  tv[par], sems[par]).wait()

        @pl.when(0 < nw)
        def _():
            wfire(0, 0)

        @pl.when(1 < nw)
        def _():
            wfire(1, 1)

        @pl.loop(0, 124, step=2)
        def _(c):
            for par in range(2):
                k = c + par

                @pl.when(k < nw)
                def _():
                    wdrain(par)
                    transpose_win(tv[par], WWIN, ov)
                    j = lo + k
                    pltpu.sync_copy(
                        ov, rm_hbm.at[pl.ds(j * (WWIN * DIM), WWIN * DIM)])

                    @pl.when(k + 2 < nw)
                    def _():
                        wfire(k + 2, par)

        @pl.when(wid == NWORK - 1)
        def _():
            pltpu.sync_copy(otail_hbm, tt)
            transpose_win(tt, 64, ov64)
            pltpu.sync_copy(ov64, rm_hbm.at[pl.ds(TAIL0 * DIM, 64 * DIM)])

    return kern(iemb_t, oemb_t, itail, otail, inputs_f)


def _sc_scores(labels_f, noise_f, irows, oemb_rm):
    mesh = plsc.VectorSubcoreMesh(core_axis_name="c", subcore_axis_name="s")

    @functools.partial(
        pl.kernel,
        compiler_params=_sc_compiler_params(False),
        out_type=jax.ShapeDtypeStruct((PAIRS * COLS,), jnp.float32),
        mesh=mesh,
        scratch_types=[
            pltpu.VMEM((BPW * DIM,), jnp.float32),
            pltpu.VMEM((CP,), jnp.int32),
            pltpu.VMEM((CP,), jnp.int32),
            pltpu.VMEM((CP, DIM), jnp.float32),
            pltpu.VMEM((CP, DIM), jnp.float32),
            pltpu.VMEM((CP * NSAMP,), jnp.int32),
            pltpu.VMEM((CP * NSAMP,), jnp.int32),
            pltpu.VMEM((NSAMP * CP,), jnp.int32),
            pltpu.VMEM((NSAMP * CP,), jnp.int32),
            pltpu.VMEM((NSAMP * CP, DIM), jnp.float32),
            pltpu.VMEM((NSAMP * CP, DIM), jnp.float32),
            pltpu.VMEM((CP * COLS,), jnp.float32),
            pltpu.VMEM((CP * COLS,), jnp.float32),
            pltpu.SemaphoreType.DMA,
            pltpu.SemaphoreType.DMA,
        ],
    )
    def kern(labels_hbm, noise_hbm, irows_hbm, oemb_hbm, scores_hbm,
             inp_rows, lab0, lab1, out0, out1, nlin0, nlin1,
             nt0, nt1, nr0, nr1, sc0, sc1, sem0, sem1):
        lab = (lab0, lab1)
        outr = (out0, out1)
        nlin = (nlin0, nlin1)
        nt = (nt0, nt1)
        nrows = (nr0, nr1)
        scv = (sc0, sc1)
        sems = (sem0, sem1)

        wid = lax.axis_index("s") * NCORES + lax.axis_index("c")
        wp0 = wid * PW
        iota = lax.iota(jnp.int32, LANES)

        pltpu.sync_copy(irows_hbm.at[pl.ds(wid * (BPW * DIM), BPW * DIM)],
                        inp_rows)

        def stage(c, bi):
            bp = wp0 + c * CP
            pltpu.sync_copy(labels_hbm.at[pl.ds(bp, CP)], lab[bi])
            pltpu.sync_copy(noise_hbm.at[pl.ds(bp * NSAMP, CP * NSAMP)],
                            nlin[bi])
            for g in range(CP // LANES):
                rowbase = (iota + g * LANES) * NSAMP
                for s in range(NSAMP):
                    v = plsc.load_gather(nlin[bi], [rowbase + s])
                    nt[bi][pl.ds(s * CP + g * LANES, LANES)] = v
            pltpu.async_copy(oemb_hbm.at[lab[bi]], outr[bi], sems[bi])
            for k in range(NGATH):
                pltpu.async_copy(
                    oemb_hbm.at[nt[bi].at[pl.ds(k * GROWS, GROWS)]],
                    nrows[bi].at[pl.ds(k * GROWS, GROWS)], sems[bi])

        def wait_chunk(bi):
            pltpu.make_async_copy(oemb_hbm.at[pl.ds(0, CP)], outr[bi],
                                  sems[bi]).wait()
            for k in range(NGATH):
                pltpu.make_async_copy(
                    oemb_hbm.at[pl.ds(0, GROWS)],
                    nrows[bi].at[pl.ds(k * GROWS, GROWS)], sems[bi]).wait()

        def compute(c, bi):
            def group(g, carry):
                pch = iota + g * LANES
                bloc = (pch + c * CP) // WINDOW
                dvecs = [jnp.full((LANES,), d, jnp.int32) for d in range(DIM)]
                inpv = [plsc.load_gather(inp_rows, [bloc * DIM + d])
                        for d in range(DIM)]
                base = pch * COLS

                def dot_rows(ref, rowv):
                    a = [jnp.zeros((LANES,), jnp.float32) for _ in range(4)]
                    for db in range(0, DIM, 8):
                        ls = [plsc.load_gather(ref, [rowv, dvecs[d]])
                              for d in range(db, db + 8)]
                        for t, d in enumerate(range(db, db + 8)):
                            a[t % 4] = a[t % 4] + inpv[d] * ls[t]
                    return (a[0] + a[1]) + (a[2] + a[3])

                plsc.store_scatter(scv[bi], [base + NSAMP],
                                   dot_rows(outr[bi], pch))
                for s in range(NSAMP):
                    plsc.store_scatter(scv[bi], [base + s],
                                       -dot_rows(nrows[bi], pch + s * CP))
                zero = jnp.zeros((LANES,), jnp.float32)
                for pcol in range(NSAMP + 1, COLS):
                    plsc.store_scatter(scv[bi], [base + pcol], zero)
                return carry

            lax.fori_loop(0, CP // LANES, group, 0)

        stage(0, 0)
        stage(1, 1)

        @pl.loop(0, NCHUNK, step=2)
        def _(c):
            for par in range(2):
                cc = c + par
                wait_chunk(par)
                compute(cc, par)
                pltpu.sync_copy(scv[par],
                                scores_hbm.at[pl.ds((wp0 + cc * CP) * COLS,
                                                    CP * COLS)])

                @pl.when(cc + 2 < NCHUNK)
                def _():
                    stage(cc + 2, par)

    return kern(labels_f, noise_f, irows, oemb_rm)


def _tc_loss(scores):
    rows = PAIRS * COLS // 128
    x2 = scores.reshape(rows, 128)

    def body(s_ref, o_ref):
        x = s_ref[...]
        r = lax.broadcasted_iota(jnp.int32, x.shape, 0)
        cc = lax.broadcasted_iota(jnp.int32, x.shape, 1)
        j = (r * 128 + cc) % COLS
        val = jnp.where(j <= NSAMP, jnp.log(jax.nn.sigmoid(x)), 0.0)
        o_ref[0, 0] = -jnp.sum(val) / BATCH

    out = pl.pallas_call(
        body,
        out_shape=jax.ShapeDtypeStruct((1, 1), jnp.float32),
        out_specs=pl.BlockSpec(memory_space=pltpu.SMEM),
    )(x2)
    return out[0, 0]


def kernel(inputs, labels, num_sampled, input_embed, out_embed, noise_idx):
    inputs_f = inputs.reshape(-1).astype(jnp.int32)
    labels_f = labels.reshape(-1).astype(jnp.int32)
    noise_f = noise_idx.reshape(-1).astype(jnp.int32)
    iemb_t = jnp.transpose(input_embed)
    oemb_t = jnp.transpose(out_embed)
    itail = lax.slice(iemb_t, (0, TAIL0), (DIM, VOCAB))
    otail = lax.slice(oemb_t, (0, TAIL0), (DIM, VOCAB))
    oemb_rm, irows = _sc_prep(iemb_t, oemb_t, itail, otail, inputs_f)
    scores = _sc_scores(labels_f, noise_f, irows,
                        oemb_rm.reshape(VOCAB, DIM))
    return _tc_loss(scores)


# speedup vs baseline: 5.3201x; 2.3210x over previous
import dataclasses
import functools

import jax
import jax.numpy as jnp
from jax import lax
from jax.experimental import pallas as pl
from jax.experimental.pallas import tpu as pltpu
from jax.experimental.pallas import tpu_sc as plsc

VOCAB = 1000000
DIM = 32
BATCH = 1024
WINDOW = 20
NSAMP = 20

NCORES = 2
NSUB = 16
LANES = 16
NWORK = NCORES * NSUB
PAIRS = BATCH * WINDOW
PW = PAIRS // NWORK
CP = 64
NCHUNK = PW // CP
GROWS = 128
NGATH = CP * NSAMP // GROWS
BPW = BATCH // NWORK
COLS = 24

WWIN = 256
NWIN = (VOCAB - 64) // WWIN
TAIL0 = NWIN * WWIN


def _sc_compiler_params(tc_tiling):
    cp = pltpu.CompilerParams()
    if "needs_layout_passes" in pltpu.CompilerParams.__dataclass_fields__:
        cp = dataclasses.replace(cp, needs_layout_passes=False)
    if "use_tc_tiling_on_sc" in pltpu.CompilerParams.__dataclass_fields__:
        cp = dataclasses.replace(cp, use_tc_tiling_on_sc=tc_tiling)
    return cp


def _sc_prep(iemb_t, oemb_t, itail, otail, inputs_f):
    mesh = plsc.VectorSubcoreMesh(core_axis_name="c", subcore_axis_name="s")

    @functools.partial(
        pl.kernel,
        compiler_params=_sc_compiler_params(True),
        out_type=(jax.ShapeDtypeStruct((VOCAB * DIM,), jnp.float32),
                  jax.ShapeDtypeStruct((BATCH * DIM,), jnp.float32)),
        mesh=mesh,
        scratch_types=[
            pltpu.VMEM((DIM, WWIN), jnp.float32),
            pltpu.VMEM((DIM, WWIN), jnp.float32),
            pltpu.VMEM((WWIN * DIM,), jnp.float32),
            pltpu.VMEM((DIM, 64), jnp.float32),
            pltpu.VMEM((64 * DIM,), jnp.float32),
            pltpu.VMEM((BPW * DIM,), jnp.float32),
            pltpu.VMEM((BPW,), jnp.int32),
            pltpu.SemaphoreType.DMA,
            pltpu.SemaphoreType.DMA,
        ],
    )
    def kern(iemb_hbm, oemb_hbm, itail_hbm, otail_hbm, inputs_hbm,
             rm_hbm, irows_hbm,
             tv0, tv1, ov, tt, ov64, ivals, idxs, sem0, sem1):
        tv = (tv0, tv1)
        sems = (sem0, sem1)
        wid = lax.axis_index("s") * NCORES + lax.axis_index("c")
        iota = lax.iota(jnp.int32, LANES)

        def transpose_win(src, width, dst):
            @plsc.parallel_loop(0, width // LANES, unroll=2)
            def _(g):
                lanev = iota + g * LANES
                lanev_d = lanev * DIM
                rot0 = lanev & (DIM - 1)
                for db in range(0, DIM, 8):
                    vs = [src[d, pl.ds(g * LANES, LANES)]
                          for d in range(db, db + 8)]
                    for t, d in enumerate(range(db, db + 8)):
                        plsc.store_scatter(
                            dst, [lanev_d + ((rot0 + d) & (DIM - 1))], vs[t])

        pltpu.sync_copy(inputs_hbm.at[pl.ds(wid * BPW, BPW)], idxs)

        def iread(i):
            s16 = pl.multiple_of((i // LANES) * LANES, 8)
            v = idxs[pl.ds(s16, LANES)]
            return jnp.sum(jnp.where(iota == i % LANES, v, 0))

        def ifire(i, par):
            r = iread(i)
            base = jnp.minimum((r // 128) * 128, TAIL0 - WWIN + 64)
            base = pl.multiple_of(base, 128)
            pltpu.async_copy(iemb_hbm.at[:, pl.ds(base, WWIN)],
                             tv[par], sems[par])

        def idrain(par):
            pltpu.make_async_copy(iemb_hbm.at[:, pl.ds(0, WWIN)],
                                  tv[par], sems[par]).wait()

        ifire(0, 0)
        ifire(1, 1)

        @pl.loop(0, BPW, step=2)
        def _(c):
            for par in range(2):
                i = c + par
                idrain(par)
                r = iread(i)
                ibase = i * DIM

                @pl.when(r < TAIL0)
                def _():
                    base = jnp.minimum((r // 128) * 128, TAIL0 - WWIN + 64)
                    lane = jnp.full((LANES,), 0, jnp.int32) + (r - base)
                    for h in range(2):
                        v = plsc.load_gather(
                            tv[par], [iota + h * LANES, lane])
                        plsc.store_scatter(
                            ivals, [(iota + h * LANES) + ibase], v)

                @pl.when(r >= TAIL0)
                def _():
                    pltpu.sync_copy(itail_hbm, tt)
                    lane = jnp.full((LANES,), 0, jnp.int32) + (r - TAIL0)
                    for h in range(2):
                        v = plsc.load_gather(tt, [iota + h * LANES, lane])
                        plsc.store_scatter(
                            ivals, [(iota + h * LANES) + ibase], v)

                @pl.when(i + 2 < BPW)
                def _():
                    ifire(i + 2, par)

        pltpu.sync_copy(ivals,
                        irows_hbm.at[pl.ds(wid * (BPW * DIM), BPW * DIM)])

        nw = jnp.where(wid < 2, 123, 122)
        lo = wid * 122 + jnp.minimum(wid, 2)

        def wfire(k, par):
            j = lo + k
            pltpu.async_copy(oemb_hbm.at[:, pl.ds(j * WWIN, WWIN)],
                             tv[par], sems[par])

        def wdrain(par):
            pltpu.make_async_copy(oemb_hbm.at[:, pl.ds(0, WWIN)],
                                  tv[par], sems[par]).wait()

        @pl.when(0 < nw)
        def _():
            wfire(0, 0)

        @pl.when(1 < nw)
        def _():
            wfire(1, 1)

        @pl.loop(0, 124, step=2)
        def _(c):
            for par in range(2):
                k = c + par

                @pl.when(k < nw)
                def _():
                    wdrain(par)
                    transpose_win(tv[par], WWIN, ov)
                    j = lo + k
                    pltpu.sync_copy(
                        ov, rm_hbm.at[pl.ds(j * (WWIN * DIM), WWIN * DIM)])

                    @pl.when(k + 2 < nw)
                    def _():
                        wfire(k + 2, par)

        @pl.when(wid == NWORK - 1)
        def _():
            pltpu.sync_copy(otail_hbm, tt)
            transpose_win(tt, 64, ov64)
            pltpu.sync_copy(ov64, rm_hbm.at[pl.ds(TAIL0 * DIM, 64 * DIM)])

    return kern(iemb_t, oemb_t, itail, otail, inputs_f)


def _sc_scores(labels_f, noise_f, irows, oemb_rm):
    mesh = plsc.VectorSubcoreMesh(core_axis_name="c", subcore_axis_name="s")

    @functools.partial(
        pl.kernel,
        compiler_params=_sc_compiler_params(False),
        out_type=jax.ShapeDtypeStruct((PAIRS * COLS,), jnp.float32),
        mesh=mesh,
        scratch_types=[
            pltpu.VMEM((BPW * DIM,), jnp.float32),
            pltpu.VMEM((CP,), jnp.int32),
            pltpu.VMEM((CP,), jnp.int32),
            pltpu.VMEM((CP, DIM), jnp.float32),
            pltpu.VMEM((CP, DIM), jnp.float32),
            pltpu.VMEM((CP * NSAMP,), jnp.int32),
            pltpu.VMEM((CP * NSAMP,), jnp.int32),
            pltpu.VMEM((NSAMP * CP,), jnp.int32),
            pltpu.VMEM((NSAMP * CP,), jnp.int32),
            pltpu.VMEM((NSAMP * CP, DIM), jnp.float32),
            pltpu.VMEM((NSAMP * CP, DIM), jnp.float32),
            pltpu.VMEM((CP * COLS,), jnp.float32),
            pltpu.VMEM((CP * COLS,), jnp.float32),
            pltpu.SemaphoreType.DMA,
            pltpu.SemaphoreType.DMA,
        ],
    )
    def kern(labels_hbm, noise_hbm, irows_hbm, oemb_hbm, scores_hbm,
             inp_rows, lab0, lab1, out0, out1, nlin0, nlin1,
             nt0, nt1, nr0, nr1, sc0, sc1, sem0, sem1):
        lab = (lab0, lab1)
        outr = (out0, out1)
        nlin = (nlin0, nlin1)
        nt = (nt0, nt1)
        nrows = (nr0, nr1)
        scv = (sc0, sc1)
        sems = (sem0, sem1)

        wid = lax.axis_index("s") * NCORES + lax.axis_index("c")
        wp0 = wid * PW
        iota = lax.iota(jnp.int32, LANES)

        pltpu.sync_copy(irows_hbm.at[pl.ds(wid * (BPW * DIM), BPW * DIM)],
                        inp_rows)

        def stage(c, bi):
            bp = wp0 + c * CP
            pltpu.sync_copy(labels_hbm.at[pl.ds(bp, CP)], lab[bi])
            pltpu.sync_copy(noise_hbm.at[pl.ds(bp * NSAMP, CP * NSAMP)],
                            nlin[bi])
            for g in range(CP // LANES):
                rowbase = (iota + g * LANES) * NSAMP
                for s in range(NSAMP):
                    v = plsc.load_gather(nlin[bi], [rowbase + s])
                    nt[bi][pl.ds(s * CP + g * LANES, LANES)] = v
            pltpu.async_copy(oemb_hbm.at[lab[bi]], outr[bi], sems[bi])
            for k in range(NGATH):
                pltpu.async_copy(
                    oemb_hbm.at[nt[bi].at[pl.ds(k * GROWS, GROWS)]],
                    nrows[bi].at[pl.ds(k * GROWS, GROWS)], sems[bi])

        def wait_chunk(bi):
            pltpu.make_async_copy(oemb_hbm.at[pl.ds(0, CP)], outr[bi],
                                  sems[bi]).wait()
            for k in range(NGATH):
                pltpu.make_async_copy(
                    oemb_hbm.at[pl.ds(0, GROWS)],
                    nrows[bi].at[pl.ds(k * GROWS, GROWS)], sems[bi]).wait()

        def compute(c, bi):
            def group(g, carry):
                pch = iota + g * LANES
                bloc = (pch + c * CP) // WINDOW
                inpv = [plsc.load_gather(inp_rows, [bloc * DIM + d])
                        for d in range(DIM)]
                base = pch * COLS

                def dot_rows(ref, rowv, rot0):
                    a = [jnp.zeros((LANES,), jnp.float32) for _ in range(4)]
                    for db in range(0, DIM, 8):
                        ls = [plsc.load_gather(
                                  ref, [rowv, (rot0 + d) & (DIM - 1)])
                              for d in range(db, db + 8)]
                        for t, d in enumerate(range(db, db + 8)):
                            a[t % 4] = a[t % 4] + inpv[d] * ls[t]
                    return (a[0] + a[1]) + (a[2] + a[3])

                lrot = plsc.load_gather(lab[bi], [pch]) & (DIM - 1)
                plsc.store_scatter(scv[bi], [base + NSAMP],
                                   dot_rows(outr[bi], pch, lrot))
                for s in range(NSAMP):
                    nrot = plsc.load_gather(nt[bi],
                                            [pch + s * CP]) & (DIM - 1)
                    plsc.store_scatter(
                        scv[bi], [base + s],
                        -dot_rows(nrows[bi], pch + s * CP, nrot))
                zero = jnp.zeros((LANES,), jnp.float32)
                for pcol in range(NSAMP + 1, COLS):
                    plsc.store_scatter(scv[bi], [base + pcol], zero)
                return carry

            lax.fori_loop(0, CP // LANES, group, 0)

        stage(0, 0)
        stage(1, 1)

        @pl.loop(0, NCHUNK, step=2)
        def _(c):
            for par in range(2):
                cc = c + par
                wait_chunk(par)
                compute(cc, par)
                pltpu.sync_copy(scv[par],
                                scores_hbm.at[pl.ds((wp0 + cc * CP) * COLS,
                                                    CP * COLS)])

                @pl.when(cc + 2 < NCHUNK)
                def _():
                    stage(cc + 2, par)

    return kern(labels_f, noise_f, irows, oemb_rm)


def _tc_loss(scores):
    rows = PAIRS * COLS // 128
    x2 = scores.reshape(rows, 128)

    def body(s_ref, o_ref):
        x = s_ref[...]
        r = lax.broadcasted_iota(jnp.int32, x.shape, 0)
        cc = lax.broadcasted_iota(jnp.int32, x.shape, 1)
        j = (r * 128 + cc) % COLS
        val = jnp.where(j <= NSAMP, jnp.log(jax.nn.sigmoid(x)), 0.0)
        o_ref[0, 0] = -jnp.sum(val) / BATCH

    out = pl.pallas_call(
        body,
        out_shape=jax.ShapeDtypeStruct((1, 1), jnp.float32),
        out_specs=pl.BlockSpec(memory_space=pltpu.SMEM),
    )(x2)
    return out[0, 0]


def kernel(inputs, labels, num_sampled, input_embed, out_embed, noise_idx):
    inputs_f = inputs.reshape(-1).astype(jnp.int32)
    labels_f = labels.reshape(-1).astype(jnp.int32)
    noise_f = noise_idx.reshape(-1).astype(jnp.int32)
    iemb_t = jnp.transpose(input_embed)
    oemb_t = jnp.transpose(out_embed)
    itail = lax.slice(iemb_t, (0, TAIL0), (DIM, VOCAB))
    otail = lax.slice(oemb_t, (0, TAIL0), (DIM, VOCAB))
    oemb_rm, irows = _sc_prep(iemb_t, oemb_t, itail, otail, inputs_f)
    scores = _sc_scores(labels_f, noise_f, irows,
                        oemb_rm.reshape(VOCAB, DIM))
    return _tc_loss(scores)
